# Initial kernel scaffold; baseline (speedup 1.0000x reference)
import jax
import jax.numpy as jnp
from jax.experimental import pallas as pl


def kernel(x_BSD, Wg_DN, Wl1_ND2H, Wl2_NHD):
    raise NotImplementedError("write your pallas kernel here")


# speedup vs baseline: 1.1371x; 1.1371x over previous
import functools

import jax
import jax.numpy as jnp
from jax.experimental import pallas as pl
from jax.experimental.pallas import tpu as pltpu

D = 768
H = 768
N = 8
NG = 7
M = 2048
TM = 256
T = M // TM
NITEMS = T + NG - 1


def _shared_router_kernel(x_ref, wg_ref, w1_ref, w2_ref, ids_ref, ysh_ref):
    x = x_ref[...]
    logits = jnp.dot(x, wg_ref[...], preferred_element_type=jnp.float32)
    cols = jax.lax.broadcasted_iota(jnp.int32, logits.shape, 1)
    ml = jnp.where(cols < NG, logits, -jnp.inf)
    mx = jnp.max(ml, axis=1, keepdims=True)
    ids = jnp.min(jnp.where(ml >= mx, cols, NG), axis=1)
    ids_ref[0, 0, :] = ids.astype(jnp.int32)
    z = jnp.dot(x, w1_ref[0], preferred_element_type=jnp.float32)
    a = z[:, :H]
    b = z[:, H:]
    h = a * b * jax.nn.sigmoid(b)
    ysh_ref[...] = jnp.dot(h, w2_ref[0], preferred_element_type=jnp.float32)


def _gmm_kernel(meta_ref, xs_ref, w1_ref, w2_ref, out_ref):
    w = pl.program_id(0)
    lo = meta_ref[2, w]
    hi = meta_ref[3, w]
    init = meta_ref[4, w]

    @pl.when(init == 1)
    def _():
        out_ref[...] = jnp.zeros_like(out_ref)

    @pl.when(lo < hi)
    def _():
        x = xs_ref[...]
        z = jnp.dot(x, w1_ref[0], preferred_element_type=jnp.float32)
        a = z[:, :H]
        b = z[:, H:]
        h = a * b * jax.nn.sigmoid(b)
        o = jnp.dot(h, w2_ref[0], preferred_element_type=jnp.float32)
        rows = jax.lax.broadcasted_iota(jnp.int32, (TM, 1), 0)
        keep = (rows >= lo) & (rows < hi)
        out_ref[...] += jnp.where(keep, o, 0.0)


@functools.partial(jax.jit, static_argnames=("interpret",))
def _run(x_BSD, Wg_DN, Wl1_ND2H, Wl2_NHD, interpret=False):
    x = x_BSD.reshape(M, D)

    ids3, y_sh = pl.pallas_call(
        _shared_router_kernel,
        grid=(T,),
        in_specs=[
            pl.BlockSpec((TM, D), lambda t: (t, 0)),
            pl.BlockSpec((D, N), lambda t: (0, 0)),
            pl.BlockSpec((1, D, 2 * H), lambda t: (NG, 0, 0)),
            pl.BlockSpec((1, H, D), lambda t: (NG, 0, 0)),
        ],
        out_specs=[
            pl.BlockSpec((1, 1, TM), lambda t: (t, 0, 0)),
            pl.BlockSpec((TM, D), lambda t: (t, 0)),
        ],
        out_shape=[
            jax.ShapeDtypeStruct((T, 1, TM), jnp.int32),
            jax.ShapeDtypeStruct((M, D), jnp.float32),
        ],
        interpret=interpret,
    )(x, Wg_DN, Wl1_ND2H, Wl2_NHD)
    ids = ids3.reshape(M)

    sort_idx = jnp.argsort(ids, stable=True)
    xs = jnp.take(x, sort_idx, axis=0)
    c = jnp.sum(ids[None, :] < jnp.arange(NG + 1)[:, None], axis=1)

    tt = jnp.arange(T * NG, dtype=jnp.int32) // NG
    ee = jnp.arange(T * NG, dtype=jnp.int32) % NG
    ce = c[ee]
    ce1 = c[ee + 1]
    touched = (ce < (tt + 1) * TM) & (ce1 > tt * TM)
    pos = jnp.where(touched, jnp.cumsum(touched) - 1, NITEMS)
    item_tile = jnp.full((NITEMS,), T - 1, jnp.int32).at[pos].set(tt, mode="drop")
    item_exp = jnp.full((NITEMS,), NG - 1, jnp.int32).at[pos].set(ee, mode="drop")
    item_lo = jnp.zeros((NITEMS,), jnp.int32).at[pos].set(
        jnp.maximum(ce - tt * TM, 0).astype(jnp.int32), mode="drop")
    item_hi = jnp.zeros((NITEMS,), jnp.int32).at[pos].set(
        jnp.minimum(ce1 - tt * TM, TM).astype(jnp.int32), mode="drop")
    item_init = jnp.concatenate(
        [jnp.ones((1,), jnp.int32),
         (item_tile[1:] != item_tile[:-1]).astype(jnp.int32)])
    meta = jnp.stack([item_tile, item_exp, item_lo, item_hi, item_init])

    y_rs = pl.pallas_call(
        _gmm_kernel,
        grid_spec=pltpu.PrefetchScalarGridSpec(
            num_scalar_prefetch=1,
            grid=(NITEMS,),
            in_specs=[
                pl.BlockSpec((TM, D), lambda w, m: (m[0, w], 0)),
                pl.BlockSpec((1, D, 2 * H), lambda w, m: (m[1, w], 0, 0)),
                pl.BlockSpec((1, H, D), lambda w, m: (m[1, w], 0, 0)),
            ],
            out_specs=pl.BlockSpec((TM, D), lambda w, m: (m[0, w], 0)),
        ),
        out_shape=jax.ShapeDtypeStruct((M, D), jnp.float32),
        interpret=interpret,
    )(meta, xs, Wl1_ND2H, Wl2_NHD)

    y = y_sh.at[sort_idx].add(y_rs)
    return y.reshape(x_BSD.shape)


def kernel(x_BSD, Wg_DN, Wl1_ND2H, Wl2_NHD):
    return _run(x_BSD, Wg_DN, Wl1_ND2H, Wl2_NHD)


# speedup vs baseline: 1.8121x; 1.5936x over previous
import functools

import jax
import jax.numpy as jnp
from jax.experimental import pallas as pl
from jax.experimental.pallas import tpu as pltpu

D = 768
H = 768
N = 8
NG = 7
M = 2048
TM = 256
T = M // TM
NITEMS = T + NG - 1


def _shared_router_kernel(x_ref, wg_ref, w1_ref, w2_ref, ids_ref, ysh_ref):
    x = x_ref[...]
    logits = jnp.dot(x, wg_ref[...], preferred_element_type=jnp.float32)
    cols = jax.lax.broadcasted_iota(jnp.int32, logits.shape, 1)
    ml = jnp.where(cols < NG, logits, -jnp.inf)
    mx = jnp.max(ml, axis=1, keepdims=True)
    ids = jnp.min(jnp.where(ml >= mx, cols, NG), axis=1)
    ids_ref[0, 0, :] = ids.astype(jnp.int32)
    z = jnp.dot(x, w1_ref[0], preferred_element_type=jnp.float32)
    a = z[:, :H]
    b = z[:, H:]
    h = a * b * jax.nn.sigmoid(b)
    ysh_ref[...] = jnp.dot(h, w2_ref[0], preferred_element_type=jnp.float32)


def _gmm_kernel(meta_ref, xs_ref, w1_ref, w2_ref, out_ref):
    w = pl.program_id(0)
    lo = meta_ref[2, w]
    hi = meta_ref[3, w]
    init = meta_ref[4, w]

    @pl.when(init == 1)
    def _():
        out_ref[...] = jnp.zeros_like(out_ref)

    @pl.when(lo < hi)
    def _():
        x = xs_ref[...]
        z = jnp.dot(x, w1_ref[0], preferred_element_type=jnp.float32)
        a = z[:, :H]
        b = z[:, H:]
        h = a * b * jax.nn.sigmoid(b)
        o = jnp.dot(h, w2_ref[0], preferred_element_type=jnp.float32)
        rows = jax.lax.broadcasted_iota(jnp.int32, (TM, 1), 0)
        keep = (rows >= lo) & (rows < hi)
        out_ref[...] += jnp.where(keep, o, 0.0)


@functools.partial(jax.jit, static_argnames=("interpret",))
def _run(x_BSD, Wg_DN, Wl1_ND2H, Wl2_NHD, interpret=False):
    x = x_BSD.reshape(M, D)

    ids3, y_sh = pl.pallas_call(
        _shared_router_kernel,
        grid=(T,),
        in_specs=[
            pl.BlockSpec((TM, D), lambda t: (t, 0)),
            pl.BlockSpec((D, N), lambda t: (0, 0)),
            pl.BlockSpec((1, D, 2 * H), lambda t: (NG, 0, 0)),
            pl.BlockSpec((1, H, D), lambda t: (NG, 0, 0)),
        ],
        out_specs=[
            pl.BlockSpec((1, 1, TM), lambda t: (t, 0, 0)),
            pl.BlockSpec((TM, D), lambda t: (t, 0)),
        ],
        out_shape=[
            jax.ShapeDtypeStruct((T, 1, TM), jnp.int32),
            jax.ShapeDtypeStruct((M, D), jnp.float32),
        ],
        interpret=interpret,
    )(x, Wg_DN, Wl1_ND2H, Wl2_NHD)
    ids = ids3.reshape(M)

    sort_idx = jnp.argsort(ids, stable=True)
    xs = jnp.take(x, sort_idx, axis=0)
    c = jnp.sum(ids[None, :] < jnp.arange(NG + 1)[:, None], axis=1)

    pp = jnp.arange(T * NG, dtype=jnp.int32)
    tt = pp // NG
    ee = pp % NG
    ce = c[ee]
    ce1 = c[ee + 1]
    touched = (ce < (tt + 1) * TM) & (ce1 > tt * TM)
    order = jnp.argsort(jnp.where(touched, pp, T * NG))[:NITEMS]
    n_items = jnp.sum(touched.astype(jnp.int32))
    real = jnp.arange(NITEMS, dtype=jnp.int32) < n_items
    item_tile = jnp.where(real, tt[order], T - 1)
    item_exp = jnp.where(real, ee[order], NG - 1)
    item_lo = jnp.where(real, jnp.maximum(ce - tt * TM, 0)[order], 0).astype(jnp.int32)
    item_hi = jnp.where(real, jnp.minimum(ce1 - tt * TM, TM)[order], 0).astype(jnp.int32)
    item_init = jnp.concatenate(
        [jnp.ones((1,), jnp.int32),
         (item_tile[1:] != item_tile[:-1]).astype(jnp.int32)])
    meta = jnp.stack([item_tile, item_exp, item_lo, item_hi, item_init])

    y_rs = pl.pallas_call(
        _gmm_kernel,
        grid_spec=pltpu.PrefetchScalarGridSpec(
            num_scalar_prefetch=1,
            grid=(NITEMS,),
            in_specs=[
                pl.BlockSpec((TM, D), lambda w, m: (m[0, w], 0)),
                pl.BlockSpec((1, D, 2 * H), lambda w, m: (m[1, w], 0, 0)),
                pl.BlockSpec((1, H, D), lambda w, m: (m[1, w], 0, 0)),
            ],
            out_specs=pl.BlockSpec((TM, D), lambda w, m: (m[0, w], 0)),
        ),
        out_shape=jax.ShapeDtypeStruct((M, D), jnp.float32),
        interpret=interpret,
    )(meta, xs, Wl1_ND2H, Wl2_NHD)

    inv = jnp.argsort(sort_idx)
    y = y_sh + jnp.take(y_rs, inv, axis=0)
    return y.reshape(x_BSD.shape)


def kernel(x_BSD, Wg_DN, Wl1_ND2H, Wl2_NHD):
    return _run(x_BSD, Wg_DN, Wl1_ND2H, Wl2_NHD)
